# baseline (speedup 1.0000x reference)
import functools
import math

import jax
import jax.numpy as jnp
from jax import lax
from jax.experimental import pallas as pl
from jax.experimental.pallas import tpu as pltpu
from jax.experimental.pallas import tpu_sc as plsc

TRUNC_R = 0.85
NEG = -70.0
NITER = 16
LN2 = 0.6931471805599453

B, C, P = 16, 2888, 1024
L = 16
NW = 32
GPB = P // L
JOBS = B * GPB
JPW = JOBS // NW
UNROLL = 8
CU = C // UNROLL


def _sc_compute(xbuf, qbuf, out_done_wait):
    zeros = jnp.zeros((L,), jnp.float32)

    def mx_body(i, accs):
        a0, a1, a2, a3 = accs
        base = i * UNROLL
        for k in range(0, UNROLL, 4):
            a0 = jnp.maximum(a0, xbuf[base + k])
            a1 = jnp.maximum(a1, xbuf[base + k + 1])
            a2 = jnp.maximum(a2, xbuf[base + k + 2])
            a3 = jnp.maximum(a3, xbuf[base + k + 3])
        return a0, a1, a2, a3
    m0, m1, m2, m3 = lax.fori_loop(
        0, CU, mx_body, (jnp.full((L,), -1e30, jnp.float32),) * 4)
    m = jnp.maximum(jnp.maximum(m0, m1), jnp.maximum(m2, m3))

    out_done_wait()

    def eq_body(i, accs):
        a0, a1 = accs
        base = i * UNROLL
        for k in range(0, UNROLL, 2):
            q0 = jnp.exp(xbuf[base + k] - m)
            q1 = jnp.exp(xbuf[base + k + 1] - m)
            qbuf[base + k] = q0
            qbuf[base + k + 1] = q1
            a0 = a0 + q0
            a1 = a1 + q1
        return a0, a1
    s0, s1 = lax.fori_loop(0, CU, eq_body, (zeros, zeros))
    s = s0 + s1
    rs = TRUNC_R * s

    bits = plsc.bitcast(s, jnp.int32)
    e = lax.shift_right_logical(bits, 23) - 127
    y = e.astype(jnp.float32) * LN2
    for _ in range(4):
        y = y - 1.0 + s * jnp.exp(-y)

    lo0 = y + math.log((1.0 - TRUNC_R) / C)
    hi0 = zeros

    def iter_body(_, carry):
        lo, hi = carry
        mid = 0.5 * (lo + hi)
        thr = jnp.exp(mid)

        def ms_body(i, accs):
            a0, a1, a2, a3 = accs
            base = i * UNROLL
            for k in range(0, UNROLL, 4):
                q0 = qbuf[base + k]
                q1 = qbuf[base + k + 1]
                q2 = qbuf[base + k + 2]
                q3 = qbuf[base + k + 3]
                a0 = a0 + jnp.where(q0 > thr, q0, 0.0)
                a1 = a1 + jnp.where(q1 > thr, q1, 0.0)
                a2 = a2 + jnp.where(q2 > thr, q2, 0.0)
                a3 = a3 + jnp.where(q3 > thr, q3, 0.0)
            return a0, a1, a2, a3
        g = lax.fori_loop(0, CU, ms_body, (zeros,) * 4)
        mass = (g[0] + g[1]) + (g[2] + g[3])
        pred = mass >= rs
        lo = jnp.where(pred, mid, lo)
        hi = jnp.where(pred, hi, mid)
        return lo, hi

    lo, hi = lax.fori_loop(0, NITER, iter_body, (lo0, hi0))
    thr = jnp.exp(lo)
    moff = m + y

    def fin_body(i, _):
        base = i * UNROLL
        for k in range(UNROLL):
            xc = xbuf[base + k]
            qc = qbuf[base + k]
            lx = jnp.minimum(jnp.maximum(xc - moff, NEG), 0.0)
            qbuf[base + k] = jnp.where(qc > thr, lx, NEG)
        return 0
    lax.fori_loop(0, CU, fin_body, 0)


def _tc_body(x_ref, o_ref):
    x = x_ref[0]
    m = jnp.max(x, axis=0, keepdims=True)
    q = jnp.exp(x - m)
    s = jnp.sum(q, axis=0, keepdims=True)
    rs = TRUNC_R * s
    lo = (1.0 - TRUNC_R) / x.shape[0] * s
    hi = jnp.ones_like(s)
    for _ in range(16):
        mid = jnp.sqrt(lo * hi)
        mass = jnp.sum(jnp.where(q > mid, q, 0.0), axis=0, keepdims=True)
        pred = mass >= rs
        lo = jnp.where(pred, mid, lo)
        hi = jnp.where(pred, hi, mid)
    logx = jnp.clip(x - (m + jnp.log(s)), NEG, 0.0)
    o_ref[0] = jnp.where(q > lo, logx, NEG)


BT = 11


def _make_sc_body(n_batches, b_off):
    gpb = P // L
    jobs = n_batches * gpb
    jpw = jobs // NW

    def body(x_hbm, out_hbm, xbuf, qbuf, sem_in, sem_out):
        cid = lax.axis_index("c")
        sid = lax.axis_index("s")
        wid = sid * 2 + cid

        def src(jid):
            b = jid // gpb
            p0 = (jid % gpb) * L
            return x_hbm.at[b + b_off, :, pl.ds(p0, L)]

        def dst(jid):
            b = jid // gpb
            p0 = (jid % gpb) * L
            return out_hbm.at[b, :, pl.ds(p0, L)]

        first = wid * jpw
        pltpu.async_copy(src(first), xbuf, sem_in)

        def job(j, _):
            jid = first + j
            pltpu.make_async_copy(src(jid), xbuf, sem_in).wait()

            def out_done_wait():
                @pl.when(j > 0)
                def _():
                    pltpu.make_async_copy(qbuf, dst(jid), sem_out).wait()

            _sc_compute(xbuf, qbuf, out_done_wait)
            pltpu.async_copy(qbuf, dst(jid), sem_out)

            @pl.when(j + 1 < jpw)
            def _():
                pltpu.async_copy(src(jid + 1), xbuf, sem_in)
            return 0

        lax.fori_loop(0, jpw, job, 0)
        pltpu.make_async_copy(qbuf, dst(first), sem_out).wait()

    return body


@jax.jit
def kernel(logits):
    sc_call = functools.partial(
        pl.kernel,
        mesh=plsc.VectorSubcoreMesh(core_axis_name="c", subcore_axis_name="s"),
        out_type=jax.ShapeDtypeStruct((B - BT, C, P), jnp.float32),
        scratch_types=[
            pltpu.VMEM((C, L), jnp.float32),
            pltpu.VMEM((C, L), jnp.float32),
            pltpu.SemaphoreType.DMA,
            pltpu.SemaphoreType.DMA,
        ],
        compiler_params=pltpu.CompilerParams(
            use_tc_tiling_on_sc=False, needs_layout_passes=False),
    )(_make_sc_body(B - BT, BT))
    sc_out = sc_call(logits)

    W = 512
    tc_out = pl.pallas_call(
        _tc_body,
        grid=(BT, P // W),
        in_specs=[pl.BlockSpec((1, C, W), lambda b, p: (b, 0, p))],
        out_specs=pl.BlockSpec((1, C, W), lambda b, p: (b, 0, p)),
        out_shape=jax.ShapeDtypeStruct((B, C, P), jnp.float32),
    )(logits)

    return lax.dynamic_update_slice(tc_out, sc_out, (BT, 0, 0))


# speedup vs baseline: 1.0876x; 1.0876x over previous
import functools
import math

import jax
import jax.numpy as jnp
from jax import lax
from jax.experimental import pallas as pl
from jax.experimental.pallas import tpu as pltpu
from jax.experimental.pallas import tpu_sc as plsc

TRUNC_R = 0.85
NEG = -70.0
NITER = 16
LN2 = 0.6931471805599453

B, C, P = 16, 2888, 1024
L = 16
NW = 32
GPB = P // L
JOBS = B * GPB
JPW = JOBS // NW
UNROLL = 8
CU = C // UNROLL


def _sc_compute(xbuf, qbuf, out_done_wait):
    zeros = jnp.zeros((L,), jnp.float32)

    def mx_body(i, accs):
        a0, a1, a2, a3 = accs
        base = i * UNROLL
        for k in range(0, UNROLL, 4):
            a0 = jnp.maximum(a0, xbuf[base + k])
            a1 = jnp.maximum(a1, xbuf[base + k + 1])
            a2 = jnp.maximum(a2, xbuf[base + k + 2])
            a3 = jnp.maximum(a3, xbuf[base + k + 3])
        return a0, a1, a2, a3
    m0, m1, m2, m3 = lax.fori_loop(
        0, CU, mx_body, (jnp.full((L,), -1e30, jnp.float32),) * 4)
    m = jnp.maximum(jnp.maximum(m0, m1), jnp.maximum(m2, m3))

    out_done_wait()

    def eq_body(i, accs):
        a0, a1 = accs
        base = i * UNROLL
        for k in range(0, UNROLL, 2):
            q0 = jnp.exp(xbuf[base + k] - m)
            q1 = jnp.exp(xbuf[base + k + 1] - m)
            qbuf[base + k] = q0
            qbuf[base + k + 1] = q1
            a0 = a0 + q0
            a1 = a1 + q1
        return a0, a1
    s0, s1 = lax.fori_loop(0, CU, eq_body, (zeros, zeros))
    s = s0 + s1
    rs = TRUNC_R * s

    bits = plsc.bitcast(s, jnp.int32)
    e = lax.shift_right_logical(bits, 23) - 127
    y = e.astype(jnp.float32) * LN2
    for _ in range(4):
        y = y - 1.0 + s * jnp.exp(-y)

    lo0 = y + math.log((1.0 - TRUNC_R) / C)
    hi0 = zeros

    def iter_body(_, carry):
        lo, hi = carry
        mid = 0.5 * (lo + hi)
        thr = jnp.exp(mid)

        def ms_body(i, accs):
            a0, a1, a2, a3 = accs
            base = i * UNROLL
            for k in range(0, UNROLL, 4):
                q0 = qbuf[base + k]
                q1 = qbuf[base + k + 1]
                q2 = qbuf[base + k + 2]
                q3 = qbuf[base + k + 3]
                a0 = a0 + jnp.where(q0 > thr, q0, 0.0)
                a1 = a1 + jnp.where(q1 > thr, q1, 0.0)
                a2 = a2 + jnp.where(q2 > thr, q2, 0.0)
                a3 = a3 + jnp.where(q3 > thr, q3, 0.0)
            return a0, a1, a2, a3
        g = lax.fori_loop(0, CU, ms_body, (zeros,) * 4)
        mass = (g[0] + g[1]) + (g[2] + g[3])
        pred = mass >= rs
        lo = jnp.where(pred, mid, lo)
        hi = jnp.where(pred, hi, mid)
        return lo, hi

    lo, hi = lax.fori_loop(0, NITER, iter_body, (lo0, hi0))
    thr = jnp.exp(lo)
    moff = m + y

    def fin_body(i, _):
        base = i * UNROLL
        for k in range(UNROLL):
            xc = xbuf[base + k]
            qc = qbuf[base + k]
            lx = jnp.minimum(jnp.maximum(xc - moff, NEG), 0.0)
            qbuf[base + k] = jnp.where(qc > thr, lx, NEG)
        return 0
    lax.fori_loop(0, CU, fin_body, 0)


def _tc_body(x_ref, o_ref):
    x = x_ref[0]
    m = jnp.max(x, axis=0, keepdims=True)
    q = jnp.exp(x - m)
    s = jnp.sum(q, axis=0, keepdims=True)
    rs = TRUNC_R * s
    lo = (1.0 - TRUNC_R) / x.shape[0] * s
    hi = jnp.ones_like(s)
    for _ in range(16):
        mid = jnp.sqrt(lo * hi)
        mass = jnp.sum(jnp.where(q > mid, q, 0.0), axis=0, keepdims=True)
        pred = mass >= rs
        lo = jnp.where(pred, mid, lo)
        hi = jnp.where(pred, hi, mid)
    logx = jnp.clip(x - (m + jnp.log(s)), NEG, 0.0)
    o_ref[0] = jnp.where(q > lo, logx, NEG)


BT = 12


def _make_sc_body(n_batches, b_off):
    gpb = P // L
    jobs = n_batches * gpb
    jpw = jobs // NW

    def body(x_hbm, out_hbm, xbuf, qbuf, sem_in, sem_out):
        cid = lax.axis_index("c")
        sid = lax.axis_index("s")
        wid = sid * 2 + cid

        def src(jid):
            b = jid // gpb
            p0 = (jid % gpb) * L
            return x_hbm.at[b + b_off, :, pl.ds(p0, L)]

        def dst(jid):
            b = jid // gpb
            p0 = (jid % gpb) * L
            return out_hbm.at[b, :, pl.ds(p0, L)]

        first = wid * jpw
        pltpu.async_copy(src(first), xbuf, sem_in)

        def job(j, _):
            jid = first + j
            pltpu.make_async_copy(src(jid), xbuf, sem_in).wait()

            def out_done_wait():
                @pl.when(j > 0)
                def _():
                    pltpu.make_async_copy(qbuf, dst(jid), sem_out).wait()

            _sc_compute(xbuf, qbuf, out_done_wait)
            pltpu.async_copy(qbuf, dst(jid), sem_out)

            @pl.when(j + 1 < jpw)
            def _():
                pltpu.async_copy(src(jid + 1), xbuf, sem_in)
            return 0

        lax.fori_loop(0, jpw, job, 0)
        pltpu.make_async_copy(qbuf, dst(first), sem_out).wait()

    return body


@jax.jit
def kernel(logits):
    sc_call = functools.partial(
        pl.kernel,
        mesh=plsc.VectorSubcoreMesh(core_axis_name="c", subcore_axis_name="s"),
        out_type=jax.ShapeDtypeStruct((B - BT, C, P), jnp.float32),
        scratch_types=[
            pltpu.VMEM((C, L), jnp.float32),
            pltpu.VMEM((C, L), jnp.float32),
            pltpu.SemaphoreType.DMA,
            pltpu.SemaphoreType.DMA,
        ],
        compiler_params=pltpu.CompilerParams(
            use_tc_tiling_on_sc=False, needs_layout_passes=False),
    )(_make_sc_body(B - BT, BT))
    sc_out = sc_call(logits)

    W = 512
    tc_out = pl.pallas_call(
        _tc_body,
        grid=(BT, P // W),
        in_specs=[pl.BlockSpec((1, C, W), lambda b, p: (b, 0, p))],
        out_specs=pl.BlockSpec((1, C, W), lambda b, p: (b, 0, p)),
        out_shape=jax.ShapeDtypeStruct((B, C, P), jnp.float32),
    )(logits)

    return lax.dynamic_update_slice(tc_out, sc_out, (BT, 0, 0))
